# baseline (speedup 1.0000x reference)
import jax
import jax.numpy as jnp
from jax.experimental import pallas as pl

INP_DIM = 128
HID_DIM = 16
NUM_FACTORS = 8
ROUTIT = 6
TAU = 1.0


def _init_body(x_ref, wm_ref, mask_ref, out_ref):
    x = x_ref[...]
    wm = wm_ref[...]
    z = jnp.maximum(jnp.dot(x, wm, preferred_element_type=jnp.float32), 0.0)
    m = mask_ref[...]
    sq = z * z
    n2 = jnp.dot(sq, m, preferred_element_type=jnp.float32)
    inv = jax.lax.rsqrt(jnp.maximum(n2, 1e-24))
    inv = jnp.where(jnp.sqrt(n2) < 1e-12, 0.0, inv)
    invb = jnp.dot(inv, m.T, preferred_element_type=jnp.float32)
    out_ref[...] = z * invb


def _disen_init(X, W):
    n = X.shape[0]
    wm = W.transpose(2, 0, 1).reshape(INP_DIM, NUM_FACTORS * HID_DIM)
    mask = jnp.repeat(jnp.eye(NUM_FACTORS, dtype=jnp.float32), HID_DIM, axis=0)
    blk = 1000
    return pl.pallas_call(
        _init_body,
        grid=(n // blk,),
        in_specs=[
            pl.BlockSpec((blk, INP_DIM), lambda i: (i, 0)),
            pl.BlockSpec((INP_DIM, NUM_FACTORS * HID_DIM), lambda i: (0, 0)),
            pl.BlockSpec((NUM_FACTORS * HID_DIM, NUM_FACTORS), lambda i: (0, 0)),
        ],
        out_specs=pl.BlockSpec((blk, NUM_FACTORS * HID_DIM), lambda i: (i, 0)),
        out_shape=jax.ShapeDtypeStruct((n, NUM_FACTORS * HID_DIM), jnp.float32),
    )(X, wm, mask)


def _l2norm(x, axis):
    n = jnp.linalg.norm(x, ord=2, axis=axis, keepdims=True)
    return x / jnp.maximum(n, 1e-12)


def kernel(X, edges, W, b):
    n = X.shape[0]
    z = _disen_init(X, W).reshape(n, NUM_FACTORS, HID_DIM)
    src = edges[0]
    trg = edges[1]
    zz = z
    c = z
    for t in range(ROUTIT):
        p = jnp.sum(zz[src] * c[trg], axis=2, keepdims=True)
        p = jax.nn.softmax(p / TAU, axis=1)
        weight_sum = p * zz[trg]
        c_new = c.at[src].add(weight_sum)
        if t == 0:
            zz = c_new
        c = _l2norm(c_new, 2)
    return c.reshape(n, -1)


# speedup vs baseline: 36.5329x; 36.5329x over previous
import functools

import jax
import jax.numpy as jnp
from jax import lax
from jax.experimental import pallas as pl
from jax.experimental.pallas import tpu as pltpu
from jax.experimental.pallas import tpu_sc as plsc

INP_DIM = 128
HID_DIM = 16
NUM_FACTORS = 8
ROUTIT = 6
TAU = 1.0
FDIM = NUM_FACTORS * HID_DIM

NTILES = 32
CHUNK = 64
ROWS_PER_SUB = None


def _init_body(x_ref, wm_ref, mask_ref, out_ref):
    x = x_ref[...]
    wm = wm_ref[...]
    z = jnp.maximum(jnp.dot(x, wm, preferred_element_type=jnp.float32), 0.0)
    m = mask_ref[...]
    n2 = jnp.dot(z * z, m, preferred_element_type=jnp.float32)
    inv = 1.0 / jnp.maximum(jnp.sqrt(n2), 1e-12)
    invb = jnp.dot(inv, m.T, preferred_element_type=jnp.float32)
    out_ref[...] = z * invb


def _disen_init(X, W):
    n = X.shape[0]
    wm = W.transpose(2, 0, 1).reshape(INP_DIM, FDIM)
    mask = jnp.repeat(jnp.eye(NUM_FACTORS, dtype=jnp.float32), HID_DIM, axis=0)
    blk = n // 16
    return pl.pallas_call(
        _init_body,
        grid=(16,),
        in_specs=[
            pl.BlockSpec((blk, INP_DIM), lambda i: (i, 0)),
            pl.BlockSpec((INP_DIM, FDIM), lambda i: (0, 0)),
            pl.BlockSpec((FDIM, NUM_FACTORS), lambda i: (0, 0)),
        ],
        out_specs=pl.BlockSpec((blk, FDIM), lambda i: (i, 0)),
        out_shape=jax.ShapeDtypeStruct((n, FDIM), jnp.float32),
    )(X, wm, mask)


_GDN = lax.GatherDimensionNumbers(
    offset_dims=(), collapsed_slice_dims=(0,), start_index_map=(0,))


def _shuf(v, idx):
    return lax.gather(v, idx.reshape(16, 1), _GDN, (1,),
                      mode=lax.GatherScatterMode.PROMISE_IN_BOUNDS)


def _route_body(first_iter, n_nodes, srcp, trgp, validp, zz_hbm, c_hbm,
                out_hbm, idxs_v, idxt_v, val_v, zs_v, ct_v, zt_v, contrib_v,
                acc_sh, sem):
    cid = lax.axis_index("c")
    sid = lax.axis_index("s")
    gtile = cid * 16 + sid
    rows_per_sub = n_nodes // 16
    ept = srcp.shape[0] // NTILES
    nchunks = ept // CHUNK
    lanes = lax.iota(jnp.int32, 16)

    zero16 = jnp.zeros((16,), jnp.float32)
    val_v[pl.ds(CHUNK, 16)] = zero16

    def _zero_row(r, _):
        for k in range(NUM_FACTORS):
            contrib_v[r, pl.ds(16 * k, 16)] = zero16
        return 0

    lax.fori_loop(0, CHUNK, _zero_row, 0, unroll=False)
    nzc = rows_per_sub // CHUNK
    rem = rows_per_sub - nzc * CHUNK
    for i in range(nzc):
        pltpu.sync_copy(contrib_v,
                        acc_sh.at[pl.ds(sid * rows_per_sub + i * CHUNK, CHUNK)])
    if rem:
        pltpu.sync_copy(contrib_v.at[pl.ds(0, rem)],
                        acc_sh.at[pl.ds(sid * rows_per_sub + nzc * CHUNK, rem)])
    plsc.subcore_barrier()

    def _chunk(j, _):
        base = gtile * ept + j * CHUNK
        pltpu.sync_copy(srcp.at[pl.ds(base, CHUNK)], idxs_v)
        pltpu.sync_copy(trgp.at[pl.ds(base, CHUNK)], idxt_v)
        pltpu.sync_copy(validp.at[pl.ds(base, CHUNK)], val_v.at[pl.ds(0, CHUNK)])
        pltpu.async_copy(zz_hbm.at[idxs_v], zs_v, sem).wait()
        pltpu.async_copy(c_hbm.at[idxt_v], ct_v, sem).wait()
        if not first_iter:
            pltpu.async_copy(zz_hbm.at[idxt_v], zt_v, sem).wait()

        def _edge(e, _c):
            ct_rows = [ct_v[e, pl.ds(16 * k, 16)] for k in range(NUM_FACTORS)]
            if first_iter:
                w_rows = ct_rows
            else:
                w_rows = [zt_v[e, pl.ds(16 * k, 16)] for k in range(NUM_FACTORS)]
            vecs = [zs_v[e, pl.ds(16 * k, 16)] * ct_rows[k]
                    for k in range(NUM_FACTORS)]
            for sh in (1, 2, 4):
                m = (lanes & sh) == 0
                vecs = [jnp.where(m,
                                  vecs[j] + _shuf(vecs[j], lanes ^ sh),
                                  vecs[j + 1] + _shuf(vecs[j + 1], lanes ^ sh))
                        for j in range(0, len(vecs), 2)]
            r = vecs[0]
            dvec = r + _shuf(r, lanes ^ 8)
            mx = dvec
            for sh in (1, 2, 4):
                mx = jnp.maximum(mx, _shuf(mx, lanes ^ sh))
            ex = jnp.exp(dvec - mx)
            ssum = ex
            for sh in (1, 2, 4):
                ssum = ssum + _shuf(ssum, lanes ^ sh)
            vv = val_v[pl.ds(e, 16)]
            pv = ex * (vv[0] / ssum)
            for k in range(NUM_FACTORS):
                pk = _shuf(pv, jnp.full((16,), k, jnp.int32))
                contrib_v[e, pl.ds(16 * k, 16)] = pk * w_rows[k]
            return _c

        lax.fori_loop(0, CHUNK, _edge, 0, unroll=False)
        pltpu.sync_copy(contrib_v, acc_sh.at[idxs_v], add=True)
        return _

    lax.fori_loop(0, nchunks, _chunk, 0, unroll=False)
    plsc.subcore_barrier()

    for i in range(nzc):
        r0 = sid * rows_per_sub + i * CHUNK
        pltpu.sync_copy(acc_sh.at[pl.ds(r0, CHUNK)], out_hbm.at[cid, pl.ds(r0, CHUNK)])
    if rem:
        r0 = sid * rows_per_sub + nzc * CHUNK
        pltpu.sync_copy(acc_sh.at[pl.ds(r0, rem)], out_hbm.at[cid, pl.ds(r0, rem)])


def _route_sc(srcp, trgp, validp, zz, c, first_iter):
    n = c.shape[0]
    mesh = plsc.VectorSubcoreMesh(core_axis_name="c", subcore_axis_name="s")
    body = functools.partial(_route_body, first_iter, n)
    return pl.kernel(
        body,
        out_type=jax.ShapeDtypeStruct((2, n, FDIM), jnp.float32),
        mesh=mesh,
        scratch_types=[
            pltpu.VMEM((CHUNK,), jnp.int32),
            pltpu.VMEM((CHUNK,), jnp.int32),
            pltpu.VMEM((CHUNK + 16,), jnp.float32),
            pltpu.VMEM((CHUNK, FDIM), jnp.float32),
            pltpu.VMEM((CHUNK, FDIM), jnp.float32),
            pltpu.VMEM((CHUNK, FDIM), jnp.float32),
            pltpu.VMEM((CHUNK, FDIM), jnp.float32),
            pltpu.VMEM_SHARED((n, FDIM), jnp.float32),
            pltpu.SemaphoreType.DMA,
        ],
        name="route_sc0" if first_iter else "route_sc",
    )(srcp, trgp, validp, zz, c)


def _combine_body(emit_zz, p_ref, c_ref, mask_ref, out_ref, zz_ref=None):
    s = p_ref[0] + p_ref[1] + c_ref[...]
    if emit_zz:
        zz_ref[...] = s
    m = mask_ref[...]
    n2 = jnp.dot(s * s, m, preferred_element_type=jnp.float32)
    inv = 1.0 / jnp.maximum(jnp.sqrt(n2), 1e-12)
    out_ref[...] = s * jnp.dot(inv, m.T, preferred_element_type=jnp.float32)


def _combine_tc(partials, c, emit_zz):
    n = c.shape[0]
    mask = jnp.repeat(jnp.eye(NUM_FACTORS, dtype=jnp.float32), HID_DIM, axis=0)
    blk = n // 16
    out_shape = [jax.ShapeDtypeStruct((n, FDIM), jnp.float32)]
    out_specs = [pl.BlockSpec((blk, FDIM), lambda i: (i, 0))]
    if emit_zz:
        out_shape.append(jax.ShapeDtypeStruct((n, FDIM), jnp.float32))
        out_specs.append(pl.BlockSpec((blk, FDIM), lambda i: (i, 0)))
    return pl.pallas_call(
        functools.partial(_combine_body, emit_zz),
        grid=(16,),
        in_specs=[
            pl.BlockSpec((2, blk, FDIM), lambda i: (0, i, 0)),
            pl.BlockSpec((blk, FDIM), lambda i: (i, 0)),
            pl.BlockSpec((FDIM, NUM_FACTORS), lambda i: (0, 0)),
        ],
        out_specs=out_specs,
        out_shape=out_shape,
    )(partials, c, mask)


def kernel(X, edges, W, b):
    n = X.shape[0]
    e = edges.shape[1]
    sub_rows = ((n + 15) // 16 + 7) // 8 * 8
    np_rows = 16 * sub_rows
    xp = jnp.concatenate(
        [X, jnp.zeros((np_rows - n, INP_DIM), jnp.float32)]) if np_rows != n else X
    z = _disen_init(xp, W)

    ept = -(-e // (NTILES * CHUNK)) * CHUNK
    epad = NTILES * ept - e
    src = edges[0]
    trg = edges[1]
    srcp = jnp.concatenate([src, jnp.zeros((epad,), jnp.int32)])
    trgp = jnp.concatenate([trg, jnp.zeros((epad,), jnp.int32)])
    validp = jnp.concatenate(
        [jnp.ones((e,), jnp.float32), jnp.zeros((epad,), jnp.float32)])

    c = z
    zz = z
    for t in range(ROUTIT):
        partials = _route_sc(srcp, trgp, validp, zz, c, first_iter=(t == 0))
        if t == 0:
            c, zz = _combine_tc(partials, c, emit_zz=True)
        else:
            (c,) = _combine_tc(partials, c, emit_zz=False)
    return c[:n]


# speedup vs baseline: 40.5496x; 1.1099x over previous
import functools

import jax
import jax.numpy as jnp
from jax import lax
from jax.experimental import pallas as pl
from jax.experimental.pallas import tpu as pltpu
from jax.experimental.pallas import tpu_sc as plsc

INP_DIM = 128
HID_DIM = 16
NUM_FACTORS = 8
ROUTIT = 6
TAU = 1.0
FDIM = NUM_FACTORS * HID_DIM

NTILES = 32
CHUNK = 56
ROWS_PER_SUB = None


def _init_body(x_ref, wm_ref, mask_ref, out_ref):
    x = x_ref[...]
    wm = wm_ref[...]
    z = jnp.maximum(jnp.dot(x, wm, preferred_element_type=jnp.float32), 0.0)
    m = mask_ref[...]
    n2 = jnp.dot(z * z, m, preferred_element_type=jnp.float32)
    inv = 1.0 / jnp.maximum(jnp.sqrt(n2), 1e-12)
    invb = jnp.dot(inv, m.T, preferred_element_type=jnp.float32)
    out_ref[...] = z * invb


def _disen_init(X, W):
    n = X.shape[0]
    wm = W.transpose(2, 0, 1).reshape(INP_DIM, FDIM)
    mask = jnp.repeat(jnp.eye(NUM_FACTORS, dtype=jnp.float32), HID_DIM, axis=0)
    blk = n // 16
    return pl.pallas_call(
        _init_body,
        grid=(16,),
        in_specs=[
            pl.BlockSpec((blk, INP_DIM), lambda i: (i, 0)),
            pl.BlockSpec((INP_DIM, FDIM), lambda i: (0, 0)),
            pl.BlockSpec((FDIM, NUM_FACTORS), lambda i: (0, 0)),
        ],
        out_specs=pl.BlockSpec((blk, FDIM), lambda i: (i, 0)),
        out_shape=jax.ShapeDtypeStruct((n, FDIM), jnp.float32),
    )(X, wm, mask)


_GDN = lax.GatherDimensionNumbers(
    offset_dims=(), collapsed_slice_dims=(0,), start_index_map=(0,))


def _shuf(v, idx):
    return lax.gather(v, idx.reshape(16, 1), _GDN, (1,),
                      mode=lax.GatherScatterMode.PROMISE_IN_BOUNDS)


def _route_body(first_iter, n_nodes, n_edges, srcp_unused, eslab, zz_hbm, c_hbm,
                out_hbm, idx0, idx1, zs0, zs1, ct0, ct1, zt0, zt1,
                acc_sh, sem0, sem1):
    cid = lax.axis_index("c")
    sid = lax.axis_index("s")
    gtile = cid * 16 + sid
    rows_per_sub = n_nodes // 16
    nchunks = eslab.shape[1] - 1
    ept = nchunks * CHUNK
    lanes = lax.iota(jnp.int32, 16)

    zero16 = jnp.zeros((16,), jnp.float32)

    def _zero_row(r, _):
        for k in range(NUM_FACTORS):
            ct0[r, pl.ds(16 * k, 16)] = zero16
        return 0

    lax.fori_loop(0, CHUNK, _zero_row, 0, unroll=False)
    nzc = rows_per_sub // CHUNK
    rem = rows_per_sub - nzc * CHUNK
    for i in range(nzc):
        pltpu.sync_copy(ct0,
                        acc_sh.at[pl.ds(sid * rows_per_sub + i * CHUNK, CHUNK)])
    if rem:
        pltpu.sync_copy(ct0.at[pl.ds(0, rem)],
                        acc_sh.at[pl.ds(sid * rows_per_sub + nzc * CHUNK, rem)])
    plsc.subcore_barrier()

    def _issue(idxb, zsb, ctb, ztb, semp):
        ds_ = [pltpu.async_copy(zz_hbm.at[idxb.at[0]], zsb, semp),
               pltpu.async_copy(c_hbm.at[idxb.at[1]], ctb, semp)]
        if not first_iter:
            ds_.append(pltpu.async_copy(zz_hbm.at[idxb.at[1]], ztb, semp))
        return ds_

    def _drain(ds_):
        for d in ds_:
            d.wait()

    def _compute(jchunk, zs_v, ct_v, zt_v):
        gbase = gtile * ept + jchunk * CHUNK

        def _edge(e, _c):
            ct_rows = [ct_v[e, pl.ds(16 * k, 16)] for k in range(NUM_FACTORS)]
            if first_iter:
                w_rows = ct_rows
            else:
                w_rows = [zt_v[e, pl.ds(16 * k, 16)] for k in range(NUM_FACTORS)]
            vecs = [zs_v[e, pl.ds(16 * k, 16)] * ct_rows[k]
                    for k in range(NUM_FACTORS)]
            for sh in (1, 2, 4):
                m = (lanes & sh) == 0
                vecs = [jnp.where(m,
                                  vecs[j] + _shuf(vecs[j], lanes ^ sh),
                                  vecs[j + 1] + _shuf(vecs[j + 1], lanes ^ sh))
                        for j in range(0, len(vecs), 2)]
            r = vecs[0]
            dvec = r + _shuf(r, lanes ^ 8)
            mx = dvec
            for sh in (1, 2, 4):
                mx = jnp.maximum(mx, _shuf(mx, lanes ^ sh))
            ex = jnp.exp(dvec - mx)
            ssum = ex
            for sh in (1, 2, 4):
                ssum = ssum + _shuf(ssum, lanes ^ sh)
            val = jnp.where(gbase + e < n_edges, 1.0, 0.0).astype(jnp.float32)
            pv = ex * (val / ssum)
            for k in range(NUM_FACTORS):
                pk = _shuf(pv, jnp.full((16,), k, jnp.int32))
                ct_v[e, pl.ds(16 * k, 16)] = pk * w_rows[k]
            return _c

        lax.fori_loop(0, CHUNK, _edge, 0, unroll=False)

    pltpu.sync_copy(eslab.at[gtile, 0], idx0)
    _drain(_issue(idx0, zs0, ct0, zt0, sem0))

    def _pair(j2, _):
        a = 2 * j2
        pltpu.sync_copy(eslab.at[gtile, a + 1], idx1)
        ds_b = _issue(idx1, zs1, ct1, zt1, sem1)
        _compute(a, zs0, ct0, zt0)
        pltpu.sync_copy(ct0, acc_sh.at[idx0.at[0]], add=True)
        _drain(ds_b)
        pltpu.sync_copy(eslab.at[gtile, a + 2], idx0)
        ds_n = _issue(idx0, zs0, ct0, zt0, sem0)
        _compute(a + 1, zs1, ct1, zt1)
        pltpu.sync_copy(ct1, acc_sh.at[idx1.at[0]], add=True)
        _drain(ds_n)
        return _

    lax.fori_loop(0, nchunks // 2, _pair, 0, unroll=False)
    plsc.subcore_barrier()

    for i in range(nzc):
        r0 = sid * rows_per_sub + i * CHUNK
        pltpu.sync_copy(acc_sh.at[pl.ds(r0, CHUNK)], out_hbm.at[cid, pl.ds(r0, CHUNK)])
    if rem:
        r0 = sid * rows_per_sub + nzc * CHUNK
        pltpu.sync_copy(acc_sh.at[pl.ds(r0, rem)], out_hbm.at[cid, pl.ds(r0, rem)])


def _route_sc(eslab, zz, c, n_edges, first_iter):
    n = c.shape[0]
    mesh = plsc.VectorSubcoreMesh(core_axis_name="c", subcore_axis_name="s")
    body = functools.partial(_route_body, first_iter, n, n_edges, None)
    return pl.kernel(
        body,
        out_type=jax.ShapeDtypeStruct((2, n, FDIM), jnp.float32),
        mesh=mesh,
        scratch_types=[
            pltpu.VMEM((2, CHUNK), jnp.int32),
            pltpu.VMEM((2, CHUNK), jnp.int32),
            pltpu.VMEM((CHUNK, FDIM), jnp.float32),
            pltpu.VMEM((CHUNK, FDIM), jnp.float32),
            pltpu.VMEM((CHUNK, FDIM), jnp.float32),
            pltpu.VMEM((CHUNK, FDIM), jnp.float32),
            pltpu.VMEM((CHUNK, FDIM), jnp.float32),
            pltpu.VMEM((CHUNK, FDIM), jnp.float32),
            pltpu.VMEM_SHARED((n, FDIM), jnp.float32),
            pltpu.SemaphoreType.DMA,
            pltpu.SemaphoreType.DMA,
        ],
        name="route_sc0" if first_iter else "route_sc",
    )(eslab, zz, c)


def _combine_body(emit_zz, p_ref, c_ref, mask_ref, out_ref, zz_ref=None):
    s = p_ref[0] + p_ref[1] + c_ref[...]
    if emit_zz:
        zz_ref[...] = s
    m = mask_ref[...]
    n2 = jnp.dot(s * s, m, preferred_element_type=jnp.float32)
    inv = 1.0 / jnp.maximum(jnp.sqrt(n2), 1e-12)
    out_ref[...] = s * jnp.dot(inv, m.T, preferred_element_type=jnp.float32)


def _combine_tc(partials, c, emit_zz):
    n = c.shape[0]
    mask = jnp.repeat(jnp.eye(NUM_FACTORS, dtype=jnp.float32), HID_DIM, axis=0)
    blk = n // 16
    out_shape = [jax.ShapeDtypeStruct((n, FDIM), jnp.float32)]
    out_specs = [pl.BlockSpec((blk, FDIM), lambda i: (i, 0))]
    if emit_zz:
        out_shape.append(jax.ShapeDtypeStruct((n, FDIM), jnp.float32))
        out_specs.append(pl.BlockSpec((blk, FDIM), lambda i: (i, 0)))
    return pl.pallas_call(
        functools.partial(_combine_body, emit_zz),
        grid=(16,),
        in_specs=[
            pl.BlockSpec((2, blk, FDIM), lambda i: (0, i, 0)),
            pl.BlockSpec((blk, FDIM), lambda i: (i, 0)),
            pl.BlockSpec((FDIM, NUM_FACTORS), lambda i: (0, 0)),
        ],
        out_specs=out_specs,
        out_shape=out_shape,
    )(partials, c, mask)


def kernel(X, edges, W, b):
    n = X.shape[0]
    e = edges.shape[1]
    sub_rows = ((n + 15) // 16 + 7) // 8 * 8
    np_rows = 16 * sub_rows
    xp = jnp.concatenate(
        [X, jnp.zeros((np_rows - n, INP_DIM), jnp.float32)]) if np_rows != n else X
    z = _disen_init(xp, W)

    nck = -(-e // (NTILES * CHUNK))
    if nck % 2:
        nck += 1
    ept = nck * CHUNK
    epad = NTILES * ept - e
    src = edges[0]
    trg = edges[1]
    srcp = jnp.concatenate([src, jnp.zeros((epad,), jnp.int32)])
    trgp = jnp.concatenate([trg, jnp.zeros((epad,), jnp.int32)])
    es = jnp.stack([srcp.reshape(NTILES, nck, CHUNK),
                    trgp.reshape(NTILES, nck, CHUNK)], axis=2)
    eslab = jnp.concatenate(
        [es, jnp.zeros((NTILES, 1, 2, CHUNK), jnp.int32)], axis=1)

    c = z
    zz = z
    for t in range(ROUTIT):
        partials = _route_sc(eslab, zz, c, e, first_iter=(t == 0))
        if t == 0:
            c, zz = _combine_tc(partials, c, emit_zz=True)
        else:
            (c,) = _combine_tc(partials, c, emit_zz=False)
    return c[:n]


# speedup vs baseline: 50.3714x; 1.2422x over previous
import functools

import jax
import jax.numpy as jnp
from jax import lax
from jax.experimental import pallas as pl
from jax.experimental.pallas import tpu as pltpu
from jax.experimental.pallas import tpu_sc as plsc

INP_DIM = 128
HID_DIM = 16
NUM_FACTORS = 8
ROUTIT = 6
TAU = 1.0
FDIM = NUM_FACTORS * HID_DIM

NTILES = 32
CHUNK = 56
ROWS_PER_SUB = None


def _init_body(x_ref, wm_ref, mask_ref, out_ref):
    x = x_ref[...]
    wm = wm_ref[...]
    z = jnp.maximum(jnp.dot(x, wm, preferred_element_type=jnp.float32), 0.0)
    m = mask_ref[...]
    n2 = jnp.dot(z * z, m, preferred_element_type=jnp.float32)
    inv = 1.0 / jnp.maximum(jnp.sqrt(n2), 1e-12)
    invb = jnp.dot(inv, m.T, preferred_element_type=jnp.float32)
    out_ref[...] = z * invb


def _disen_init(X, W):
    n = X.shape[0]
    wm = W.transpose(2, 0, 1).reshape(INP_DIM, FDIM)
    mask = jnp.repeat(jnp.eye(NUM_FACTORS, dtype=jnp.float32), HID_DIM, axis=0)
    blk = n // 16
    return pl.pallas_call(
        _init_body,
        grid=(16,),
        in_specs=[
            pl.BlockSpec((blk, INP_DIM), lambda i: (i, 0)),
            pl.BlockSpec((INP_DIM, FDIM), lambda i: (0, 0)),
            pl.BlockSpec((FDIM, NUM_FACTORS), lambda i: (0, 0)),
        ],
        out_specs=pl.BlockSpec((blk, FDIM), lambda i: (i, 0)),
        out_shape=jax.ShapeDtypeStruct((n, FDIM), jnp.float32),
    )(X, wm, mask)


_GDN = lax.GatherDimensionNumbers(
    offset_dims=(), collapsed_slice_dims=(0,), start_index_map=(0,))


def _shuf(v, idx):
    return lax.gather(v, idx.reshape(16, 1), _GDN, (1,),
                      mode=lax.GatherScatterMode.PROMISE_IN_BOUNDS)


def _route_body(first_iter, n_nodes, n_edges, srcp_unused, eslab, zz_hbm, c_hbm,
                out_hbm, idx0, idx1, zs0, zs1, ct0, ct1, zt0, zt1,
                acc_sh, sem0, sem1):
    cid = lax.axis_index("c")
    sid = lax.axis_index("s")
    gtile = cid * 16 + sid
    rows_per_sub = n_nodes // 16
    nchunks = eslab.shape[1] - 1
    ept = nchunks * CHUNK
    lanes = lax.iota(jnp.int32, 16)

    zero16 = jnp.zeros((16,), jnp.float32)

    def _zero_row(r, _):
        for k in range(NUM_FACTORS):
            ct0[r, pl.ds(16 * k, 16)] = zero16
        return 0

    lax.fori_loop(0, CHUNK, _zero_row, 0, unroll=False)
    nzc = rows_per_sub // CHUNK
    rem = rows_per_sub - nzc * CHUNK
    for i in range(nzc):
        pltpu.sync_copy(ct0,
                        acc_sh.at[pl.ds(sid * rows_per_sub + i * CHUNK, CHUNK)])
    if rem:
        pltpu.sync_copy(ct0.at[pl.ds(0, rem)],
                        acc_sh.at[pl.ds(sid * rows_per_sub + nzc * CHUNK, rem)])
    plsc.subcore_barrier()

    def _issue(idxb, zsb, ctb, ztb, semp):
        ds_ = [pltpu.async_copy(zz_hbm.at[idxb.at[0]], zsb, semp),
               pltpu.async_copy(c_hbm.at[idxb.at[1]], ctb, semp)]
        if not first_iter:
            ds_.append(pltpu.async_copy(zz_hbm.at[idxb.at[1]], ztb, semp))
        return ds_

    def _drain(ds_):
        for d in ds_:
            d.wait()

    def _compute(jchunk, zs_v, ct_v, zt_v):
        gbase = gtile * ept + jchunk * CHUNK

        def _edge(e):
            ct_rows = [ct_v[e, pl.ds(16 * k, 16)] for k in range(NUM_FACTORS)]
            if first_iter:
                w_rows = ct_rows
            else:
                w_rows = [zt_v[e, pl.ds(16 * k, 16)] for k in range(NUM_FACTORS)]
            vecs = [zs_v[e, pl.ds(16 * k, 16)] * ct_rows[k]
                    for k in range(NUM_FACTORS)]
            for sh in (1, 2, 4):
                m = (lanes & sh) == 0
                vecs = [jnp.where(m,
                                  vecs[j] + _shuf(vecs[j], lanes ^ sh),
                                  vecs[j + 1] + _shuf(vecs[j + 1], lanes ^ sh))
                        for j in range(0, len(vecs), 2)]
            r = vecs[0]
            dvec = r + _shuf(r, lanes ^ 8)
            mx = dvec
            for sh in (1, 2, 4):
                mx = jnp.maximum(mx, _shuf(mx, lanes ^ sh))
            ex = jnp.exp(dvec - mx)
            ssum = ex
            for sh in (1, 2, 4):
                ssum = ssum + _shuf(ssum, lanes ^ sh)
            val = jnp.where(gbase + e < n_edges, 1.0, 0.0).astype(jnp.float32)
            pv = ex * (val / ssum)
            for k in range(NUM_FACTORS):
                pk = _shuf(pv, jnp.full((16,), k, jnp.int32))
                ct_v[e, pl.ds(16 * k, 16)] = pk * w_rows[k]

        plsc.parallel_loop(0, CHUNK, 1, unroll=4)(_edge)

    pltpu.sync_copy(eslab.at[gtile, 0], idx0)
    _drain(_issue(idx0, zs0, ct0, zt0, sem0))

    def _pair(j2, _):
        a = 2 * j2
        pltpu.sync_copy(eslab.at[gtile, a + 1], idx1)
        ds_b = _issue(idx1, zs1, ct1, zt1, sem1)
        _compute(a, zs0, ct0, zt0)
        pltpu.sync_copy(ct0, acc_sh.at[idx0.at[0]], add=True)
        _drain(ds_b)
        pltpu.sync_copy(eslab.at[gtile, a + 2], idx0)
        ds_n = _issue(idx0, zs0, ct0, zt0, sem0)
        _compute(a + 1, zs1, ct1, zt1)
        pltpu.sync_copy(ct1, acc_sh.at[idx1.at[0]], add=True)
        _drain(ds_n)
        return _

    lax.fori_loop(0, nchunks // 2, _pair, 0, unroll=False)
    plsc.subcore_barrier()

    for i in range(nzc):
        r0 = sid * rows_per_sub + i * CHUNK
        pltpu.sync_copy(acc_sh.at[pl.ds(r0, CHUNK)], out_hbm.at[cid, pl.ds(r0, CHUNK)])
    if rem:
        r0 = sid * rows_per_sub + nzc * CHUNK
        pltpu.sync_copy(acc_sh.at[pl.ds(r0, rem)], out_hbm.at[cid, pl.ds(r0, rem)])


def _route_sc(eslab, zz, c, n_edges, first_iter):
    n = c.shape[0]
    mesh = plsc.VectorSubcoreMesh(core_axis_name="c", subcore_axis_name="s")
    body = functools.partial(_route_body, first_iter, n, n_edges, None)
    return pl.kernel(
        body,
        out_type=jax.ShapeDtypeStruct((2, n, FDIM), jnp.float32),
        mesh=mesh,
        scratch_types=[
            pltpu.VMEM((2, CHUNK), jnp.int32),
            pltpu.VMEM((2, CHUNK), jnp.int32),
            pltpu.VMEM((CHUNK, FDIM), jnp.float32),
            pltpu.VMEM((CHUNK, FDIM), jnp.float32),
            pltpu.VMEM((CHUNK, FDIM), jnp.float32),
            pltpu.VMEM((CHUNK, FDIM), jnp.float32),
            pltpu.VMEM((CHUNK, FDIM), jnp.float32),
            pltpu.VMEM((CHUNK, FDIM), jnp.float32),
            pltpu.VMEM_SHARED((n, FDIM), jnp.float32),
            pltpu.SemaphoreType.DMA,
            pltpu.SemaphoreType.DMA,
        ],
        name="route_sc0" if first_iter else "route_sc",
    )(eslab, zz, c)


def _combine_body(emit_zz, p_ref, c_ref, mask_ref, out_ref, zz_ref=None):
    s = p_ref[0] + p_ref[1] + c_ref[...]
    if emit_zz:
        zz_ref[...] = s
    m = mask_ref[...]
    n2 = jnp.dot(s * s, m, preferred_element_type=jnp.float32)
    inv = 1.0 / jnp.maximum(jnp.sqrt(n2), 1e-12)
    out_ref[...] = s * jnp.dot(inv, m.T, preferred_element_type=jnp.float32)


def _combine_tc(partials, c, emit_zz):
    n = c.shape[0]
    mask = jnp.repeat(jnp.eye(NUM_FACTORS, dtype=jnp.float32), HID_DIM, axis=0)
    blk = n // 16
    out_shape = [jax.ShapeDtypeStruct((n, FDIM), jnp.float32)]
    out_specs = [pl.BlockSpec((blk, FDIM), lambda i: (i, 0))]
    if emit_zz:
        out_shape.append(jax.ShapeDtypeStruct((n, FDIM), jnp.float32))
        out_specs.append(pl.BlockSpec((blk, FDIM), lambda i: (i, 0)))
    return pl.pallas_call(
        functools.partial(_combine_body, emit_zz),
        grid=(16,),
        in_specs=[
            pl.BlockSpec((2, blk, FDIM), lambda i: (0, i, 0)),
            pl.BlockSpec((blk, FDIM), lambda i: (i, 0)),
            pl.BlockSpec((FDIM, NUM_FACTORS), lambda i: (0, 0)),
        ],
        out_specs=out_specs,
        out_shape=out_shape,
    )(partials, c, mask)


def kernel(X, edges, W, b):
    n = X.shape[0]
    e = edges.shape[1]
    sub_rows = ((n + 15) // 16 + 7) // 8 * 8
    np_rows = 16 * sub_rows
    xp = jnp.concatenate(
        [X, jnp.zeros((np_rows - n, INP_DIM), jnp.float32)]) if np_rows != n else X
    z = _disen_init(xp, W)

    nck = -(-e // (NTILES * CHUNK))
    if nck % 2:
        nck += 1
    ept = nck * CHUNK
    epad = NTILES * ept - e
    src = edges[0]
    trg = edges[1]
    srcp = jnp.concatenate([src, jnp.zeros((epad,), jnp.int32)])
    trgp = jnp.concatenate([trg, jnp.zeros((epad,), jnp.int32)])
    es = jnp.stack([srcp.reshape(NTILES, nck, CHUNK),
                    trgp.reshape(NTILES, nck, CHUNK)], axis=2)
    eslab = jnp.concatenate(
        [es, jnp.zeros((NTILES, 1, 2, CHUNK), jnp.int32)], axis=1)

    c = z
    zz = z
    for t in range(ROUTIT):
        partials = _route_sc(eslab, zz, c, e, first_iter=(t == 0))
        if t == 0:
            c, zz = _combine_tc(partials, c, emit_zz=True)
        else:
            (c,) = _combine_tc(partials, c, emit_zz=False)
    return c[:n]


# speedup vs baseline: 50.5924x; 1.0044x over previous
import functools

import jax
import jax.numpy as jnp
from jax import lax
from jax.experimental import pallas as pl
from jax.experimental.pallas import tpu as pltpu
from jax.experimental.pallas import tpu_sc as plsc

INP_DIM = 128
HID_DIM = 16
NUM_FACTORS = 8
ROUTIT = 6
TAU = 1.0
FDIM = NUM_FACTORS * HID_DIM

NTILES = 32
CHUNK = 56
ROWS_PER_SUB = None


def _init_body(x_ref, wm_ref, mask_ref, out_ref):
    x = x_ref[...]
    wm = wm_ref[...]
    z = jnp.maximum(jnp.dot(x, wm, preferred_element_type=jnp.float32), 0.0)
    m = mask_ref[...]
    n2 = jnp.dot(z * z, m, preferred_element_type=jnp.float32)
    inv = 1.0 / jnp.maximum(jnp.sqrt(n2), 1e-12)
    invb = jnp.dot(inv, m.T, preferred_element_type=jnp.float32)
    out_ref[...] = z * invb


def _disen_init(X, W):
    n = X.shape[0]
    wm = W.transpose(2, 0, 1).reshape(INP_DIM, FDIM)
    mask = jnp.repeat(jnp.eye(NUM_FACTORS, dtype=jnp.float32), HID_DIM, axis=0)
    blk = n // 16
    return pl.pallas_call(
        _init_body,
        grid=(16,),
        in_specs=[
            pl.BlockSpec((blk, INP_DIM), lambda i: (i, 0)),
            pl.BlockSpec((INP_DIM, FDIM), lambda i: (0, 0)),
            pl.BlockSpec((FDIM, NUM_FACTORS), lambda i: (0, 0)),
        ],
        out_specs=pl.BlockSpec((blk, FDIM), lambda i: (i, 0)),
        out_shape=jax.ShapeDtypeStruct((n, FDIM), jnp.float32),
    )(X, wm, mask)


_GDN = lax.GatherDimensionNumbers(
    offset_dims=(), collapsed_slice_dims=(0,), start_index_map=(0,))


def _shuf(v, idx):
    return lax.gather(v, idx.reshape(16, 1), _GDN, (1,),
                      mode=lax.GatherScatterMode.PROMISE_IN_BOUNDS)


def _route_body(first_iter, n_nodes, n_edges, srcp_unused, eslab, zz_hbm, c_hbm,
                out_hbm, idx0, idx1, zs0, zs1, ct0, ct1, zt0, zt1,
                acc_sh, sem0, sem1):
    cid = lax.axis_index("c")
    sid = lax.axis_index("s")
    gtile = cid * 16 + sid
    rows_per_sub = n_nodes // 16
    nchunks = eslab.shape[1] - 1
    ept = nchunks * CHUNK
    lanes = lax.iota(jnp.int32, 16)

    zero16 = jnp.zeros((16,), jnp.float32)

    def _zero_row(r, _):
        for k in range(NUM_FACTORS):
            ct0[r, pl.ds(16 * k, 16)] = zero16
        return 0

    lax.fori_loop(0, CHUNK, _zero_row, 0, unroll=False)
    nzc = rows_per_sub // CHUNK
    rem = rows_per_sub - nzc * CHUNK
    for i in range(nzc):
        pltpu.sync_copy(ct0,
                        acc_sh.at[pl.ds(sid * rows_per_sub + i * CHUNK, CHUNK)])
    if rem:
        pltpu.sync_copy(ct0.at[pl.ds(0, rem)],
                        acc_sh.at[pl.ds(sid * rows_per_sub + nzc * CHUNK, rem)])
    plsc.subcore_barrier()

    def _issue(idxb, zsb, ctb, ztb, semp):
        ds_ = [pltpu.async_copy(zz_hbm.at[idxb.at[0]], zsb, semp),
               pltpu.async_copy(c_hbm.at[idxb.at[1]], ctb, semp)]
        if not first_iter:
            ds_.append(pltpu.async_copy(zz_hbm.at[idxb.at[1]], ztb, semp))
        return ds_

    def _drain(ds_):
        for d in ds_:
            d.wait()

    def _compute(jchunk, zs_v, ct_v, zt_v):
        gbase = gtile * ept + jchunk * CHUNK

        def _edge(e):
            ct_rows = [ct_v[e, pl.ds(16 * k, 16)] for k in range(NUM_FACTORS)]
            if first_iter:
                w_rows = ct_rows
            else:
                w_rows = [zt_v[e, pl.ds(16 * k, 16)] for k in range(NUM_FACTORS)]
            vecs = [zs_v[e, pl.ds(16 * k, 16)] * ct_rows[k]
                    for k in range(NUM_FACTORS)]
            for sh in (1, 2, 4):
                m = (lanes & sh) == 0
                vecs = [jnp.where(m,
                                  vecs[j] + _shuf(vecs[j], lanes ^ sh),
                                  vecs[j + 1] + _shuf(vecs[j + 1], lanes ^ sh))
                        for j in range(0, len(vecs), 2)]
            r = vecs[0]
            dvec = r + _shuf(r, lanes ^ 8)
            mx = dvec
            for sh in (1, 2, 4):
                mx = jnp.maximum(mx, _shuf(mx, lanes ^ sh))
            ex = jnp.exp(dvec - mx)
            ssum = ex
            for sh in (1, 2, 4):
                ssum = ssum + _shuf(ssum, lanes ^ sh)
            val = jnp.where(gbase + e < n_edges, 1.0, 0.0).astype(jnp.float32)
            pv = ex * (val / ssum)
            for k in range(NUM_FACTORS):
                pk = _shuf(pv, jnp.full((16,), k, jnp.int32))
                ct_v[e, pl.ds(16 * k, 16)] = pk * w_rows[k]

        plsc.parallel_loop(0, CHUNK, 1, unroll=8)(_edge)

    pltpu.sync_copy(eslab.at[gtile, 0], idx0)
    _drain(_issue(idx0, zs0, ct0, zt0, sem0))

    def _pair(j2, _):
        a = 2 * j2
        pltpu.sync_copy(eslab.at[gtile, a + 1], idx1)
        ds_b = _issue(idx1, zs1, ct1, zt1, sem1)
        _compute(a, zs0, ct0, zt0)
        pltpu.sync_copy(ct0, acc_sh.at[idx0.at[0]], add=True)
        _drain(ds_b)
        pltpu.sync_copy(eslab.at[gtile, a + 2], idx0)
        ds_n = _issue(idx0, zs0, ct0, zt0, sem0)
        _compute(a + 1, zs1, ct1, zt1)
        pltpu.sync_copy(ct1, acc_sh.at[idx1.at[0]], add=True)
        _drain(ds_n)
        return _

    lax.fori_loop(0, nchunks // 2, _pair, 0, unroll=False)
    plsc.subcore_barrier()

    for i in range(nzc):
        r0 = sid * rows_per_sub + i * CHUNK
        pltpu.sync_copy(acc_sh.at[pl.ds(r0, CHUNK)], out_hbm.at[cid, pl.ds(r0, CHUNK)])
    if rem:
        r0 = sid * rows_per_sub + nzc * CHUNK
        pltpu.sync_copy(acc_sh.at[pl.ds(r0, rem)], out_hbm.at[cid, pl.ds(r0, rem)])


def _route_sc(eslab, zz, c, n_edges, first_iter):
    n = c.shape[0]
    mesh = plsc.VectorSubcoreMesh(core_axis_name="c", subcore_axis_name="s")
    body = functools.partial(_route_body, first_iter, n, n_edges, None)
    return pl.kernel(
        body,
        out_type=jax.ShapeDtypeStruct((2, n, FDIM), jnp.float32),
        mesh=mesh,
        scratch_types=[
            pltpu.VMEM((2, CHUNK), jnp.int32),
            pltpu.VMEM((2, CHUNK), jnp.int32),
            pltpu.VMEM((CHUNK, FDIM), jnp.float32),
            pltpu.VMEM((CHUNK, FDIM), jnp.float32),
            pltpu.VMEM((CHUNK, FDIM), jnp.float32),
            pltpu.VMEM((CHUNK, FDIM), jnp.float32),
            pltpu.VMEM((CHUNK, FDIM), jnp.float32),
            pltpu.VMEM((CHUNK, FDIM), jnp.float32),
            pltpu.VMEM_SHARED((n, FDIM), jnp.float32),
            pltpu.SemaphoreType.DMA,
            pltpu.SemaphoreType.DMA,
        ],
        name="route_sc0" if first_iter else "route_sc",
    )(eslab, zz, c)


def _combine_body(emit_zz, p_ref, c_ref, mask_ref, out_ref, zz_ref=None):
    s = p_ref[0] + p_ref[1] + c_ref[...]
    if emit_zz:
        zz_ref[...] = s
    m = mask_ref[...]
    n2 = jnp.dot(s * s, m, preferred_element_type=jnp.float32)
    inv = 1.0 / jnp.maximum(jnp.sqrt(n2), 1e-12)
    out_ref[...] = s * jnp.dot(inv, m.T, preferred_element_type=jnp.float32)


def _combine_tc(partials, c, emit_zz):
    n = c.shape[0]
    mask = jnp.repeat(jnp.eye(NUM_FACTORS, dtype=jnp.float32), HID_DIM, axis=0)
    blk = n // 16
    out_shape = [jax.ShapeDtypeStruct((n, FDIM), jnp.float32)]
    out_specs = [pl.BlockSpec((blk, FDIM), lambda i: (i, 0))]
    if emit_zz:
        out_shape.append(jax.ShapeDtypeStruct((n, FDIM), jnp.float32))
        out_specs.append(pl.BlockSpec((blk, FDIM), lambda i: (i, 0)))
    return pl.pallas_call(
        functools.partial(_combine_body, emit_zz),
        grid=(16,),
        in_specs=[
            pl.BlockSpec((2, blk, FDIM), lambda i: (0, i, 0)),
            pl.BlockSpec((blk, FDIM), lambda i: (i, 0)),
            pl.BlockSpec((FDIM, NUM_FACTORS), lambda i: (0, 0)),
        ],
        out_specs=out_specs,
        out_shape=out_shape,
    )(partials, c, mask)


def kernel(X, edges, W, b):
    n = X.shape[0]
    e = edges.shape[1]
    sub_rows = ((n + 15) // 16 + 7) // 8 * 8
    np_rows = 16 * sub_rows
    xp = jnp.concatenate(
        [X, jnp.zeros((np_rows - n, INP_DIM), jnp.float32)]) if np_rows != n else X
    z = _disen_init(xp, W)

    nck = -(-e // (NTILES * CHUNK))
    if nck % 2:
        nck += 1
    ept = nck * CHUNK
    epad = NTILES * ept - e
    src = edges[0]
    trg = edges[1]
    srcp = jnp.concatenate([src, jnp.zeros((epad,), jnp.int32)])
    trgp = jnp.concatenate([trg, jnp.zeros((epad,), jnp.int32)])
    es = jnp.stack([srcp.reshape(NTILES, nck, CHUNK),
                    trgp.reshape(NTILES, nck, CHUNK)], axis=2)
    eslab = jnp.concatenate(
        [es, jnp.zeros((NTILES, 1, 2, CHUNK), jnp.int32)], axis=1)

    c = z
    zz = z
    for t in range(ROUTIT):
        partials = _route_sc(eslab, zz, c, e, first_iter=(t == 0))
        if t == 0:
            c, zz = _combine_tc(partials, c, emit_zz=True)
        else:
            (c,) = _combine_tc(partials, c, emit_zz=False)
    return c[:n]


# speedup vs baseline: 50.7221x; 1.0026x over previous
import functools

import jax
import jax.numpy as jnp
from jax import lax
from jax.experimental import pallas as pl
from jax.experimental.pallas import tpu as pltpu
from jax.experimental.pallas import tpu_sc as plsc

INP_DIM = 128
HID_DIM = 16
NUM_FACTORS = 8
ROUTIT = 6
TAU = 1.0
FDIM = NUM_FACTORS * HID_DIM

NTILES = 32
CHUNK = 56
ROWS_PER_SUB = None


def _init_body(x_ref, wm_ref, mask_ref, out_ref):
    x = x_ref[...]
    wm = wm_ref[...]
    z = jnp.maximum(jnp.dot(x, wm, preferred_element_type=jnp.float32), 0.0)
    m = mask_ref[...]
    n2 = jnp.dot(z * z, m, preferred_element_type=jnp.float32)
    inv = 1.0 / jnp.maximum(jnp.sqrt(n2), 1e-12)
    invb = jnp.dot(inv, m.T, preferred_element_type=jnp.float32)
    out_ref[...] = z * invb


def _disen_init(X, W):
    n = X.shape[0]
    wm = W.transpose(2, 0, 1).reshape(INP_DIM, FDIM)
    mask = jnp.repeat(jnp.eye(NUM_FACTORS, dtype=jnp.float32), HID_DIM, axis=0)
    blk = n // 16
    return pl.pallas_call(
        _init_body,
        grid=(16,),
        in_specs=[
            pl.BlockSpec((blk, INP_DIM), lambda i: (i, 0)),
            pl.BlockSpec((INP_DIM, FDIM), lambda i: (0, 0)),
            pl.BlockSpec((FDIM, NUM_FACTORS), lambda i: (0, 0)),
        ],
        out_specs=pl.BlockSpec((blk, FDIM), lambda i: (i, 0)),
        out_shape=jax.ShapeDtypeStruct((n, FDIM), jnp.float32),
    )(X, wm, mask)


_GDN = lax.GatherDimensionNumbers(
    offset_dims=(), collapsed_slice_dims=(0,), start_index_map=(0,))


def _shuf(v, idx):
    return lax.gather(v, idx.reshape(16, 1), _GDN, (1,),
                      mode=lax.GatherScatterMode.PROMISE_IN_BOUNDS)


def _route_body(first_iter, n_nodes, n_edges, srcp_unused, eslab, zz_hbm, c_hbm,
                out_hbm, idx0, idx1, zs0, zs1, ct0, ct1, zt0, zt1,
                acc_sh, sem0, sem1, sem_s0, sem_s1):
    cid = lax.axis_index("c")
    sid = lax.axis_index("s")
    gtile = cid * 16 + sid
    rows_per_sub = n_nodes // 16
    nchunks = eslab.shape[1] - 1
    ept = nchunks * CHUNK
    lanes = lax.iota(jnp.int32, 16)

    zero16 = jnp.zeros((16,), jnp.float32)

    def _zero_row(r, _):
        for k in range(NUM_FACTORS):
            ct0[r, pl.ds(16 * k, 16)] = zero16
            ct1[r, pl.ds(16 * k, 16)] = zero16
        return 0

    lax.fori_loop(0, CHUNK, _zero_row, 0, unroll=False)
    nzc = rows_per_sub // CHUNK
    rem = rows_per_sub - nzc * CHUNK
    for i in range(nzc):
        pltpu.sync_copy(ct0,
                        acc_sh.at[pl.ds(sid * rows_per_sub + i * CHUNK, CHUNK)])
    if rem:
        pltpu.sync_copy(ct0.at[pl.ds(0, rem)],
                        acc_sh.at[pl.ds(sid * rows_per_sub + nzc * CHUNK, rem)])
    plsc.subcore_barrier()

    def _issue(idxb, zsb, ctb, ztb, semp):
        ds_ = [pltpu.async_copy(zz_hbm.at[idxb.at[0]], zsb, semp),
               pltpu.async_copy(c_hbm.at[idxb.at[1]], ctb, semp)]
        if not first_iter:
            ds_.append(pltpu.async_copy(zz_hbm.at[idxb.at[1]], ztb, semp))
        return ds_

    def _drain(ds_):
        for d in ds_:
            d.wait()

    def _compute(jchunk, zs_v, ct_v, zt_v):
        gbase = gtile * ept + jchunk * CHUNK

        def _edge(e):
            ct_rows = [ct_v[e, pl.ds(16 * k, 16)] for k in range(NUM_FACTORS)]
            if first_iter:
                w_rows = ct_rows
            else:
                w_rows = [zt_v[e, pl.ds(16 * k, 16)] for k in range(NUM_FACTORS)]
            vecs = [zs_v[e, pl.ds(16 * k, 16)] * ct_rows[k]
                    for k in range(NUM_FACTORS)]
            for sh in (1, 2, 4):
                m = (lanes & sh) == 0
                vecs = [jnp.where(m,
                                  vecs[j] + _shuf(vecs[j], lanes ^ sh),
                                  vecs[j + 1] + _shuf(vecs[j + 1], lanes ^ sh))
                        for j in range(0, len(vecs), 2)]
            r = vecs[0]
            dvec = r + _shuf(r, lanes ^ 8)
            mx = dvec
            for sh in (1, 2, 4):
                mx = jnp.maximum(mx, _shuf(mx, lanes ^ sh))
            ex = jnp.exp(dvec - mx)
            ssum = ex
            for sh in (1, 2, 4):
                ssum = ssum + _shuf(ssum, lanes ^ sh)
            val = jnp.where(gbase + e < n_edges, 1.0, 0.0).astype(jnp.float32)
            pv = ex * (val / ssum)
            for k in range(NUM_FACTORS):
                pk = _shuf(pv, jnp.full((16,), k, jnp.int32))
                ct_v[e, pl.ds(16 * k, 16)] = pk * w_rows[k]

        plsc.parallel_loop(0, CHUNK, 1, unroll=8)(_edge)

    pltpu.sync_copy(eslab.at[gtile, 0], idx0)
    pltpu.sync_copy(eslab.at[gtile, nchunks], idx1)
    pltpu.async_copy(ct1, acc_sh.at[idx1.at[0]], sem_s1, add=True)
    _drain(_issue(idx0, zs0, ct0, zt0, sem0))

    def _pair(j2, _):
        a = 2 * j2
        pltpu.make_async_copy(ct1, acc_sh.at[idx1.at[0]], sem_s1).wait()
        pltpu.sync_copy(eslab.at[gtile, a + 1], idx1)
        ds_b = _issue(idx1, zs1, ct1, zt1, sem1)
        _compute(a, zs0, ct0, zt0)
        d_s0 = pltpu.async_copy(ct0, acc_sh.at[idx0.at[0]], sem_s0, add=True)
        _drain(ds_b)
        d_s0.wait()
        pltpu.sync_copy(eslab.at[gtile, a + 2], idx0)
        ds_n = _issue(idx0, zs0, ct0, zt0, sem0)
        _compute(a + 1, zs1, ct1, zt1)
        pltpu.async_copy(ct1, acc_sh.at[idx1.at[0]], sem_s1, add=True)
        _drain(ds_n)
        return _

    lax.fori_loop(0, nchunks // 2, _pair, 0, unroll=False)
    pltpu.make_async_copy(ct1, acc_sh.at[idx1.at[0]], sem_s1).wait()
    plsc.subcore_barrier()

    for i in range(nzc):
        r0 = sid * rows_per_sub + i * CHUNK
        pltpu.sync_copy(acc_sh.at[pl.ds(r0, CHUNK)], out_hbm.at[cid, pl.ds(r0, CHUNK)])
    if rem:
        r0 = sid * rows_per_sub + nzc * CHUNK
        pltpu.sync_copy(acc_sh.at[pl.ds(r0, rem)], out_hbm.at[cid, pl.ds(r0, rem)])


def _route_sc(eslab, zz, c, n_edges, first_iter):
    n = c.shape[0]
    mesh = plsc.VectorSubcoreMesh(core_axis_name="c", subcore_axis_name="s")
    body = functools.partial(_route_body, first_iter, n, n_edges, None)
    return pl.kernel(
        body,
        out_type=jax.ShapeDtypeStruct((2, n, FDIM), jnp.float32),
        mesh=mesh,
        scratch_types=[
            pltpu.VMEM((2, CHUNK), jnp.int32),
            pltpu.VMEM((2, CHUNK), jnp.int32),
            pltpu.VMEM((CHUNK, FDIM), jnp.float32),
            pltpu.VMEM((CHUNK, FDIM), jnp.float32),
            pltpu.VMEM((CHUNK, FDIM), jnp.float32),
            pltpu.VMEM((CHUNK, FDIM), jnp.float32),
            pltpu.VMEM((CHUNK, FDIM), jnp.float32),
            pltpu.VMEM((CHUNK, FDIM), jnp.float32),
            pltpu.VMEM_SHARED((n, FDIM), jnp.float32),
            pltpu.SemaphoreType.DMA,
            pltpu.SemaphoreType.DMA,
            pltpu.SemaphoreType.DMA,
            pltpu.SemaphoreType.DMA,
        ],
        name="route_sc0" if first_iter else "route_sc",
    )(eslab, zz, c)


def _combine_body(emit_zz, p_ref, c_ref, mask_ref, out_ref, zz_ref=None):
    s = p_ref[0] + p_ref[1] + c_ref[...]
    if emit_zz:
        zz_ref[...] = s
    m = mask_ref[...]
    n2 = jnp.dot(s * s, m, preferred_element_type=jnp.float32)
    inv = 1.0 / jnp.maximum(jnp.sqrt(n2), 1e-12)
    out_ref[...] = s * jnp.dot(inv, m.T, preferred_element_type=jnp.float32)


def _combine_tc(partials, c, emit_zz):
    n = c.shape[0]
    mask = jnp.repeat(jnp.eye(NUM_FACTORS, dtype=jnp.float32), HID_DIM, axis=0)
    blk = n // 16
    out_shape = [jax.ShapeDtypeStruct((n, FDIM), jnp.float32)]
    out_specs = [pl.BlockSpec((blk, FDIM), lambda i: (i, 0))]
    if emit_zz:
        out_shape.append(jax.ShapeDtypeStruct((n, FDIM), jnp.float32))
        out_specs.append(pl.BlockSpec((blk, FDIM), lambda i: (i, 0)))
    return pl.pallas_call(
        functools.partial(_combine_body, emit_zz),
        grid=(16,),
        in_specs=[
            pl.BlockSpec((2, blk, FDIM), lambda i: (0, i, 0)),
            pl.BlockSpec((blk, FDIM), lambda i: (i, 0)),
            pl.BlockSpec((FDIM, NUM_FACTORS), lambda i: (0, 0)),
        ],
        out_specs=out_specs,
        out_shape=out_shape,
    )(partials, c, mask)


def kernel(X, edges, W, b):
    n = X.shape[0]
    e = edges.shape[1]
    sub_rows = ((n + 15) // 16 + 7) // 8 * 8
    np_rows = 16 * sub_rows
    xp = jnp.concatenate(
        [X, jnp.zeros((np_rows - n, INP_DIM), jnp.float32)]) if np_rows != n else X
    z = _disen_init(xp, W)

    nck = -(-e // (NTILES * CHUNK))
    if nck % 2:
        nck += 1
    ept = nck * CHUNK
    epad = NTILES * ept - e
    src = edges[0]
    trg = edges[1]
    srcp = jnp.concatenate([src, jnp.zeros((epad,), jnp.int32)])
    trgp = jnp.concatenate([trg, jnp.zeros((epad,), jnp.int32)])
    es = jnp.stack([srcp.reshape(NTILES, nck, CHUNK),
                    trgp.reshape(NTILES, nck, CHUNK)], axis=2)
    eslab = jnp.concatenate(
        [es, jnp.zeros((NTILES, 1, 2, CHUNK), jnp.int32)], axis=1)

    c = z
    zz = z
    for t in range(ROUTIT):
        partials = _route_sc(eslab, zz, c, e, first_iter=(t == 0))
        if t == 0:
            c, zz = _combine_tc(partials, c, emit_zz=True)
        else:
            (c,) = _combine_tc(partials, c, emit_zz=False)
    return c[:n]


# speedup vs baseline: 53.6733x; 1.0582x over previous
import functools

import jax
import jax.numpy as jnp
from jax import lax
from jax.experimental import pallas as pl
from jax.experimental.pallas import tpu as pltpu
from jax.experimental.pallas import tpu_sc as plsc

INP_DIM = 128
HID_DIM = 16
NUM_FACTORS = 8
ROUTIT = 6
TAU = 1.0
FDIM = NUM_FACTORS * HID_DIM

NTILES = 32
CHUNK = 56
ROWS_PER_SUB = None


def _init_body(x_ref, wm_ref, mask_ref, out_ref):
    x = x_ref[...]
    wm = wm_ref[...]
    z = jnp.maximum(jnp.dot(x, wm, preferred_element_type=jnp.float32), 0.0)
    m = mask_ref[...]
    n2 = jnp.dot(z * z, m, preferred_element_type=jnp.float32)
    inv = 1.0 / jnp.maximum(jnp.sqrt(n2), 1e-12)
    invb = jnp.dot(inv, m.T, preferred_element_type=jnp.float32)
    out_ref[...] = z * invb


def _fm_mask():
    return (jnp.arange(FDIM)[:, None] % NUM_FACTORS
            == jnp.arange(NUM_FACTORS)[None, :]).astype(jnp.float32)


def _disen_init(X, W):
    n = X.shape[0]
    wm = W.transpose(1, 0, 2).reshape(FDIM, INP_DIM).T
    mask = _fm_mask()
    blk = n // 16
    return pl.pallas_call(
        _init_body,
        grid=(16,),
        in_specs=[
            pl.BlockSpec((blk, INP_DIM), lambda i: (i, 0)),
            pl.BlockSpec((INP_DIM, FDIM), lambda i: (0, 0)),
            pl.BlockSpec((FDIM, NUM_FACTORS), lambda i: (0, 0)),
        ],
        out_specs=pl.BlockSpec((blk, FDIM), lambda i: (i, 0)),
        out_shape=jax.ShapeDtypeStruct((n, FDIM), jnp.float32),
    )(X, wm, mask)


_GDN = lax.GatherDimensionNumbers(
    offset_dims=(), collapsed_slice_dims=(0,), start_index_map=(0,))


def _shuf(v, idx):
    return lax.gather(v, idx.reshape(16, 1), _GDN, (1,),
                      mode=lax.GatherScatterMode.PROMISE_IN_BOUNDS)


def _route_body(first_iter, n_nodes, n_edges, srcp_unused, eslab, zz_hbm, c_hbm,
                out_hbm, idx0, idx1, zs0, zs1, ct0, ct1, zt0, zt1,
                acc_sh, sem0, sem1, sem_s0, sem_s1):
    cid = lax.axis_index("c")
    sid = lax.axis_index("s")
    gtile = cid * 16 + sid
    rows_per_sub = n_nodes // 16
    nchunks = eslab.shape[1] - 1
    ept = nchunks * CHUNK
    lanes = lax.iota(jnp.int32, 16)

    zero16 = jnp.zeros((16,), jnp.float32)

    def _zero_row(r, _):
        for k in range(NUM_FACTORS):
            ct0[r, pl.ds(16 * k, 16)] = zero16
            ct1[r, pl.ds(16 * k, 16)] = zero16
        return 0

    lax.fori_loop(0, CHUNK, _zero_row, 0, unroll=False)
    nzc = rows_per_sub // CHUNK
    rem = rows_per_sub - nzc * CHUNK
    for i in range(nzc):
        pltpu.sync_copy(ct0,
                        acc_sh.at[pl.ds(sid * rows_per_sub + i * CHUNK, CHUNK)])
    if rem:
        pltpu.sync_copy(ct0.at[pl.ds(0, rem)],
                        acc_sh.at[pl.ds(sid * rows_per_sub + nzc * CHUNK, rem)])
    plsc.subcore_barrier()

    def _issue(idxb, zsb, ctb, ztb, semp):
        ds_ = [pltpu.async_copy(zz_hbm.at[idxb.at[0]], zsb, semp),
               pltpu.async_copy(c_hbm.at[idxb.at[1]], ctb, semp)]
        if not first_iter:
            ds_.append(pltpu.async_copy(zz_hbm.at[idxb.at[1]], ztb, semp))
        return ds_

    def _drain(ds_):
        for d in ds_:
            d.wait()

    def _compute(jchunk, zs_v, ct_v, zt_v):
        gbase = gtile * ept + jchunk * CHUNK

        def _edge(e):
            ct_rows = [ct_v[e, pl.ds(16 * k, 16)] for k in range(NUM_FACTORS)]
            if first_iter:
                w_rows = ct_rows
            else:
                w_rows = [zt_v[e, pl.ds(16 * k, 16)] for k in range(NUM_FACTORS)]
            prods = [zs_v[e, pl.ds(16 * k, 16)] * ct_rows[k]
                     for k in range(NUM_FACTORS)]
            while len(prods) > 1:
                prods = [prods[j] + prods[j + 1]
                         for j in range(0, len(prods), 2)]
            t = prods[0]
            dvec = t + _shuf(t, lanes ^ 8)
            mx = dvec
            for sh in (1, 2, 4):
                mx = jnp.maximum(mx, _shuf(mx, lanes ^ sh))
            ex = jnp.exp(dvec - mx)
            ssum = ex
            for sh in (1, 2, 4):
                ssum = ssum + _shuf(ssum, lanes ^ sh)
            val = jnp.where(gbase + e < n_edges, 1.0, 0.0).astype(jnp.float32)
            pv = ex * (val / ssum)
            for k in range(NUM_FACTORS):
                ct_v[e, pl.ds(16 * k, 16)] = pv * w_rows[k]

        plsc.parallel_loop(0, CHUNK, 1, unroll=8)(_edge)

    pltpu.sync_copy(eslab.at[gtile, 0], idx0)
    pltpu.sync_copy(eslab.at[gtile, nchunks], idx1)
    pltpu.async_copy(ct1, acc_sh.at[idx1.at[0]], sem_s1, add=True)
    _drain(_issue(idx0, zs0, ct0, zt0, sem0))

    def _pair(j2, _):
        a = 2 * j2
        pltpu.make_async_copy(ct1, acc_sh.at[idx1.at[0]], sem_s1).wait()
        pltpu.sync_copy(eslab.at[gtile, a + 1], idx1)
        ds_b = _issue(idx1, zs1, ct1, zt1, sem1)
        _compute(a, zs0, ct0, zt0)
        d_s0 = pltpu.async_copy(ct0, acc_sh.at[idx0.at[0]], sem_s0, add=True)
        _drain(ds_b)
        d_s0.wait()
        pltpu.sync_copy(eslab.at[gtile, a + 2], idx0)
        ds_n = _issue(idx0, zs0, ct0, zt0, sem0)
        _compute(a + 1, zs1, ct1, zt1)
        pltpu.async_copy(ct1, acc_sh.at[idx1.at[0]], sem_s1, add=True)
        _drain(ds_n)
        return _

    lax.fori_loop(0, nchunks // 2, _pair, 0, unroll=False)
    pltpu.make_async_copy(ct1, acc_sh.at[idx1.at[0]], sem_s1).wait()
    plsc.subcore_barrier()

    for i in range(nzc):
        r0 = sid * rows_per_sub + i * CHUNK
        pltpu.sync_copy(acc_sh.at[pl.ds(r0, CHUNK)], out_hbm.at[cid, pl.ds(r0, CHUNK)])
    if rem:
        r0 = sid * rows_per_sub + nzc * CHUNK
        pltpu.sync_copy(acc_sh.at[pl.ds(r0, rem)], out_hbm.at[cid, pl.ds(r0, rem)])


def _route_sc(eslab, zz, c, n_edges, first_iter):
    n = c.shape[0]
    mesh = plsc.VectorSubcoreMesh(core_axis_name="c", subcore_axis_name="s")
    body = functools.partial(_route_body, first_iter, n, n_edges, None)
    return pl.kernel(
        body,
        out_type=jax.ShapeDtypeStruct((2, n, FDIM), jnp.float32),
        mesh=mesh,
        scratch_types=[
            pltpu.VMEM((2, CHUNK), jnp.int32),
            pltpu.VMEM((2, CHUNK), jnp.int32),
            pltpu.VMEM((CHUNK, FDIM), jnp.float32),
            pltpu.VMEM((CHUNK, FDIM), jnp.float32),
            pltpu.VMEM((CHUNK, FDIM), jnp.float32),
            pltpu.VMEM((CHUNK, FDIM), jnp.float32),
            pltpu.VMEM((CHUNK, FDIM), jnp.float32),
            pltpu.VMEM((CHUNK, FDIM), jnp.float32),
            pltpu.VMEM_SHARED((n, FDIM), jnp.float32),
            pltpu.SemaphoreType.DMA,
            pltpu.SemaphoreType.DMA,
            pltpu.SemaphoreType.DMA,
            pltpu.SemaphoreType.DMA,
        ],
        name="route_sc0" if first_iter else "route_sc",
    )(eslab, zz, c)


def _combine_body(emit_zz, p_ref, c_ref, mask_ref, out_ref, zz_ref=None):
    s = p_ref[0] + p_ref[1] + c_ref[...]
    if emit_zz:
        zz_ref[...] = s
    m = mask_ref[...]
    n2 = jnp.dot(s * s, m, preferred_element_type=jnp.float32)
    inv = 1.0 / jnp.maximum(jnp.sqrt(n2), 1e-12)
    out_ref[...] = s * jnp.dot(inv, m.T, preferred_element_type=jnp.float32)


def _combine_tc(partials, c, emit_zz):
    n = c.shape[0]
    mask = _fm_mask()
    blk = n // 16
    out_shape = [jax.ShapeDtypeStruct((n, FDIM), jnp.float32)]
    out_specs = [pl.BlockSpec((blk, FDIM), lambda i: (i, 0))]
    if emit_zz:
        out_shape.append(jax.ShapeDtypeStruct((n, FDIM), jnp.float32))
        out_specs.append(pl.BlockSpec((blk, FDIM), lambda i: (i, 0)))
    return pl.pallas_call(
        functools.partial(_combine_body, emit_zz),
        grid=(16,),
        in_specs=[
            pl.BlockSpec((2, blk, FDIM), lambda i: (0, i, 0)),
            pl.BlockSpec((blk, FDIM), lambda i: (i, 0)),
            pl.BlockSpec((FDIM, NUM_FACTORS), lambda i: (0, 0)),
        ],
        out_specs=out_specs,
        out_shape=out_shape,
    )(partials, c, mask)


def kernel(X, edges, W, b):
    n = X.shape[0]
    e = edges.shape[1]
    sub_rows = ((n + 15) // 16 + 7) // 8 * 8
    np_rows = 16 * sub_rows
    xp = jnp.concatenate(
        [X, jnp.zeros((np_rows - n, INP_DIM), jnp.float32)]) if np_rows != n else X
    z = _disen_init(xp, W)

    nck = -(-e // (NTILES * CHUNK))
    if nck % 2:
        nck += 1
    ept = nck * CHUNK
    epad = NTILES * ept - e
    src = edges[0]
    trg = edges[1]
    srcp = jnp.concatenate([src, jnp.zeros((epad,), jnp.int32)])
    trgp = jnp.concatenate([trg, jnp.zeros((epad,), jnp.int32)])
    es = jnp.stack([srcp.reshape(NTILES, nck, CHUNK),
                    trgp.reshape(NTILES, nck, CHUNK)], axis=2)
    eslab = jnp.concatenate(
        [es, jnp.zeros((NTILES, 1, 2, CHUNK), jnp.int32)], axis=1)

    c = z
    zz = z
    for t in range(ROUTIT):
        partials = _route_sc(eslab, zz, c, e, first_iter=(t == 0))
        if t == 0:
            c, zz = _combine_tc(partials, c, emit_zz=True)
        else:
            (c,) = _combine_tc(partials, c, emit_zz=False)
    return c[:n].reshape(n, HID_DIM, NUM_FACTORS).transpose(0, 2, 1).reshape(n, FDIM)
